# Initial kernel scaffold; baseline (speedup 1.0000x reference)
import jax
import jax.numpy as jnp
from jax.experimental import pallas as pl


def kernel(x, neighbor_map, W):
    raise NotImplementedError("write your pallas kernel here")


# speedup vs baseline: 10.3437x; 10.3437x over previous
import functools

import jax
import jax.numpy as jnp
from jax import lax
from jax.experimental import pallas as pl
from jax.experimental.pallas import tpu as pltpu
from jax.experimental.pallas import tpu_sc as plsc


def _matmul_body(w_ref, xt_ref, o_ref):
    o_ref[...] = jnp.dot(w_ref[...], xt_ref[...],
                         preferred_element_type=jnp.float32)


def _gate_body(x_ref, a_ref, o_ref):
    o_ref[...] = x_ref[...] * jax.nn.sigmoid(a_ref[...])


def _make_sc_gather(K, N, NP, U):
    f32 = jnp.float32
    PT = NP // 32

    mesh = plsc.VectorSubcoreMesh(core_axis_name="c", subcore_axis_name="s")

    @functools.partial(
        pl.kernel,
        out_type=jax.ShapeDtypeStruct((NP,), f32),
        mesh=mesh,
        compiler_params=pltpu.CompilerParams(needs_layout_passes=False),
        scratch_types=[
            pltpu.VMEM((PT,), jnp.int32),
            pltpu.VMEM((PT,), f32),
            pltpu.VMEM((PT,), f32),
            pltpu.SemaphoreType.DMA,
        ],
    )
    def sc_gather(yt_hbm, nmt_hbm, s_hbm, idxb, gbuf, acc, sem):
        c = lax.axis_index("c")
        s = lax.axis_index("s")
        w = s * 2 + c
        base = w * PT

        def zv(j, carry):
            o = j * (16 * U)
            for u in range(U):
                acc[pl.ds(o + u * 16, 16)] = jnp.zeros((16,), f32)
            return carry

        lax.fori_loop(0, PT // (16 * U), zv, 0)

        def per_k(k, carry):
            pltpu.sync_copy(nmt_hbm.at[pl.ds(k * NP + base, PT)], idxb)
            koff = k * N

            def adj(j, carry2):
                o = j * (16 * U)
                for u in range(U):
                    oo = o + u * 16
                    idxb[pl.ds(oo, 16)] = idxb[pl.ds(oo, 16)] + koff
                return carry2

            lax.fori_loop(0, PT // (16 * U), adj, 0)
            pltpu.async_copy(yt_hbm.at[idxb], gbuf, sem).wait()

            def av(j, carry2):
                o = j * (16 * U)
                for u in range(U):
                    oo = o + u * 16
                    acc[pl.ds(oo, 16)] = acc[pl.ds(oo, 16)] + gbuf[pl.ds(oo, 16)]
                return carry2

            lax.fori_loop(0, PT // (16 * U), av, 0)
            return carry

        lax.fori_loop(0, K, per_k, 0)
        pltpu.sync_copy(acc, s_hbm.at[pl.ds(base, PT)])

    return sc_gather


def kernel(x, neighbor_map, W):
    N, C = x.shape
    K = neighbor_map.shape[1]
    f32 = jnp.float32

    BC = 4096
    NP = ((N + BC - 1) // BC) * BC

    Wk = W.reshape(K, C)
    xT = x.T
    nmT = jnp.pad(neighbor_map.T.astype(jnp.int32),
                  ((0, 0), (0, NP - N))).reshape(-1)

    BA = 2048
    ga = (N + BA - 1) // BA
    yt = pl.pallas_call(
        _matmul_body,
        grid=(ga,),
        in_specs=[pl.BlockSpec((K, C), lambda i: (0, 0)),
                  pl.BlockSpec((C, BA), lambda i: (0, i))],
        out_specs=pl.BlockSpec((K, BA), lambda i: (0, i)),
        out_shape=jax.ShapeDtypeStruct((K, N), f32),
    )(Wk, xT)

    sc = _make_sc_gather(K, N, NP, U=8)
    s = sc(yt.reshape(-1), nmT)

    st = s[:N].reshape(N, 1)
    BB = 2048
    gb = (N + BB - 1) // BB
    out = pl.pallas_call(
        _gate_body,
        grid=(gb,),
        in_specs=[pl.BlockSpec((BB, C), lambda i: (i, 0)),
                  pl.BlockSpec((BB, 1), lambda i: (i, 0))],
        out_specs=pl.BlockSpec((BB, C), lambda i: (i, 0)),
        out_shape=jax.ShapeDtypeStruct((N, C), f32),
    )(x, st)
    return out
